# baseline (speedup 1.0000x reference)
import functools

import jax
import jax.numpy as jnp
from jax import lax
from jax.experimental import pallas as pl
from jax.experimental.pallas import tpu as pltpu
from jax.experimental.pallas import tpu_sc as plsc

NUM_ENT = 1000000
NUM_REL = 1000
DIM = 64
BATCH = 16384

NC = 2
NS = 16
NW = NC * NS
B_PER_W = BATCH // NW
GRP = 16
NGRP = B_PER_W // GRP
NBUF = 4

NCOL = NUM_ENT // 128
TAIL = NUM_ENT - NCOL * 128
FLAT = NUM_ENT * DIM
COLS_LO = NCOL // NW
COLS_EXTRA = NCOL - COLS_LO * NW
TRIP = (COLS_LO + 2) // 2


def _transpose_block(in_buf, out_buf, b, lane, n_ent):
    bfull = jnp.full((16,), b, jnp.int32)
    for e in range(n_ent):
        efull = jnp.full((16,), e, jnp.int32)
        for c in range(DIM // 16):
            v = plsc.load_gather(in_buf, [bfull, c * 16 + lane, efull])
            out_buf[b, pl.ds(e * DIM + c * 16, 16)] = v


def _tr_body(entT_hbm, tail_hbm, flat_hbm, in_buf, out_buf, tail_v, si0, si1, so0, so1):
    wid = lax.axis_index("s") * NC + lax.axis_index("c")
    start = wid * COLS_LO + jnp.minimum(wid, COLS_EXTRA)
    ncols = COLS_LO + jnp.where(wid < COLS_EXTRA, 1, 0)
    last = start + ncols - 1
    lane = lax.iota(jnp.int32, 16)
    sin = (si0, si1)
    sout = (so0, so1)

    def col_of(i):
        return jnp.minimum(start + i, last)

    pltpu.async_copy(entT_hbm.at[:, pl.ds(col_of(0) * 128, 128)], in_buf.at[0], sin[0])

    def outer(h, _):
        for par in range(2):
            i = h * 2 + par
            pltpu.make_async_copy(entT_hbm.at[:, pl.ds(0, 128)], in_buf.at[par], sin[par]).wait()
            pltpu.async_copy(entT_hbm.at[:, pl.ds(col_of(i + 1) * 128, 128)],
                             in_buf.at[1 - par], sin[1 - par])
            @pl.when(i >= 2)
            def _():
                pltpu.make_async_copy(flat_hbm.at[pl.ds(0, 128 * DIM)],
                                      out_buf.at[par], sout[par]).wait()
            _transpose_block(in_buf, out_buf, par, lane, 128)
            pltpu.async_copy(out_buf.at[par],
                             flat_hbm.at[pl.ds(col_of(i) * (128 * DIM), 128 * DIM)],
                             sout[par])
        return 0

    lax.fori_loop(0, TRIP, outer, 0)

    pltpu.make_async_copy(flat_hbm.at[pl.ds(0, 128 * DIM)], out_buf.at[0], sout[0]).wait()
    pltpu.make_async_copy(flat_hbm.at[pl.ds(0, 128 * DIM)], out_buf.at[1], sout[1]).wait()
    pltpu.make_async_copy(entT_hbm.at[:, pl.ds(0, 128)], in_buf.at[0], sin[0]).wait()

    @pl.when(wid == NW - 1)
    def _():
        pltpu.sync_copy(tail_hbm, tail_v)
        pltpu.sync_copy(tail_v, flat_hbm.at[pl.ds(NCOL * 128 * DIM, TAIL * DIM)])


def _issue_group(g, bi, idx_s, idx_o, idx_r, flat_hbm, reltab_hbm,
                 sub_b, obj_b, rel_b, sems):
    vs = idx_s[pl.ds(g * GRP, GRP)]
    vo = idx_o[pl.ds(g * GRP, GRP)]
    vr = idx_r[pl.ds(g * GRP, GRP)]
    for l in range(GRP):
        row = pl.ds(l * DIM, DIM)
        pltpu.async_copy(flat_hbm.at[pl.ds(vs[l] * DIM, DIM)], sub_b.at[bi, row], sems[bi][0])
        pltpu.async_copy(flat_hbm.at[pl.ds(vo[l] * DIM, DIM)], obj_b.at[bi, row], sems[bi][1])
        pltpu.async_copy(reltab_hbm.at[pl.ds(vr[l], 1)], rel_b.at[bi, pl.ds(l, 1)], sems[bi][2])


def _gather_body(subj_hbm, obj_hbm, rel_hbm, flat_hbm, reltab_hbm, out_hbm,
                 idx_s, idx_o, idx_r, sub_b, obj_b, rel_b, score_v,
                 s0, s1, s2, s3, s4, s5, s6, s7, s8, s9, s10, s11):
    wid = lax.axis_index("s") * NC + lax.axis_index("c")
    sems = [(s0, s1, s2), (s3, s4, s5), (s6, s7, s8), (s9, s10, s11)]

    pltpu.sync_copy(subj_hbm.at[wid], idx_s)
    pltpu.sync_copy(obj_hbm.at[wid], idx_o)
    pltpu.sync_copy(rel_hbm.at[wid], idx_r)

    lane = lax.iota(jnp.int32, GRP)

    for g in range(NBUF - 1):
        _issue_group(g, g, idx_s, idx_o, idx_r, flat_hbm, reltab_hbm,
                     sub_b, obj_b, rel_b, sems)

    def outer(h, _):
        for p in range(NBUF):
            g = h * NBUF + p

            pltpu.make_async_copy(flat_hbm.at[pl.ds(0, GRP * DIM)],
                                  sub_b.at[p], sems[p][0]).wait()
            pltpu.make_async_copy(flat_hbm.at[pl.ds(0, GRP * DIM)],
                                  obj_b.at[p], sems[p][1]).wait()
            pltpu.make_async_copy(reltab_hbm.at[pl.ds(0, GRP)],
                                  rel_b.at[p], sems[p][2]).wait()

            nb = (p + NBUF - 1) % NBUF

            @pl.when(g + NBUF - 1 < NGRP)
            def _():
                _issue_group(g + NBUF - 1, nb, idx_s, idx_o, idx_r,
                             flat_hbm, reltab_hbm, sub_b, obj_b, rel_b, sems)

            out = jnp.zeros((GRP,), jnp.float32)
            for l in range(GRP):
                acc = jnp.zeros((16,), jnp.float32)
                for c in range(DIM // 16):
                    col = pl.ds(l * DIM + c * 16, 16)
                    d = sub_b[p, col] + rel_b[p, l, pl.ds(c * 16, 16)] - obj_b[p, col]
                    acc = acc + d * d
                s = lax.reduce_sum(acc, axes=(0,))
                out = jnp.where(lane == l, s, out)
            score_v[pl.ds(g * GRP, GRP)] = out
        return 0

    lax.fori_loop(0, NGRP // NBUF, outer, 0)

    pltpu.sync_copy(score_v, out_hbm.at[wid])


@jax.jit
def _transe(subjects, objects, relations, ent_embedding, rel_embedding):
    mesh = plsc.VectorSubcoreMesh(core_axis_name="c", subcore_axis_name="s")
    kern_t = pl.kernel(
        _tr_body,
        out_type=jax.ShapeDtypeStruct((FLAT,), jnp.float32),
        mesh=mesh,
        scratch_types=[
            pltpu.VMEM((2, DIM, 128), jnp.float32),
            pltpu.VMEM((2, 128 * DIM), jnp.float32),
            pltpu.VMEM((TAIL * DIM,), jnp.float32),
        ] + [pltpu.SemaphoreType.DMA] * 4,
        compiler_params=pltpu.CompilerParams(
            needs_layout_passes=False, use_tc_tiling_on_sc=True),
    )
    kern_g = pl.kernel(
        _gather_body,
        out_type=jax.ShapeDtypeStruct((NW, B_PER_W), jnp.float32),
        mesh=mesh,
        scratch_types=[
            pltpu.VMEM((B_PER_W,), jnp.int32),
            pltpu.VMEM((B_PER_W,), jnp.int32),
            pltpu.VMEM((B_PER_W,), jnp.int32),
            pltpu.VMEM((NBUF, GRP * DIM), jnp.float32),
            pltpu.VMEM((NBUF, GRP * DIM), jnp.float32),
            pltpu.VMEM((NBUF, GRP, DIM), jnp.float32),
            pltpu.VMEM((B_PER_W,), jnp.float32),
        ] + [pltpu.SemaphoreType.DMA] * 12,
        compiler_params=pltpu.CompilerParams(
            needs_layout_passes=False, use_tc_tiling_on_sc=True),
    )
    tail = ent_embedding[NCOL * 128:].reshape(TAIL * DIM)
    flat = kern_t(ent_embedding.T, tail)
    subj = subjects.astype(jnp.int32).reshape(NW, B_PER_W)
    obj = objects.astype(jnp.int32).reshape(NW, B_PER_W)
    rel = relations.astype(jnp.int32).reshape(NW, B_PER_W)
    out = kern_g(subj, obj, rel, flat, rel_embedding)
    return out.reshape(BATCH, 1)


def kernel(subjects, objects, relations, ent_embedding, rel_embedding):
    return _transe(subjects, objects, relations, ent_embedding, rel_embedding)


# speedup vs baseline: 1.2871x; 1.2871x over previous
import functools

import jax
import jax.numpy as jnp
from jax import lax
from jax.experimental import pallas as pl
from jax.experimental.pallas import tpu as pltpu
from jax.experimental.pallas import tpu_sc as plsc

NUM_ENT = 1000000
NUM_REL = 1000
DIM = 64
BATCH = 16384

NC = 2
NS = 16
NW = NC * NS
B_PER_W = BATCH // NW
GRP = 16
NGRP = B_PER_W // GRP
NBUF = 4

NCOL = NUM_ENT // 128
TAIL = NUM_ENT - NCOL * 128
FLAT = NUM_ENT * DIM
COLS_LO = NCOL // NW
COLS_EXTRA = NCOL - COLS_LO * NW
TRIP = (COLS_LO + 2) // 2


def _transpose_block(in_buf, out_buf, b, lane, n_ent):
    bfull = jnp.full((16,), b, jnp.int32)
    for e0 in range(0, n_ent, 2):
        vals = []
        for e in (e0, e0 + 1):
            efull = jnp.full((16,), e, jnp.int32)
            for c in range(DIM // 16):
                vals.append(plsc.load_gather(in_buf, [bfull, c * 16 + lane, efull]))
        for k, v in enumerate(vals):
            e = e0 + k // 4
            c = k % 4
            out_buf[b, pl.ds(e * DIM + c * 16, 16)] = v


def _tr_body(entT_hbm, tail_hbm, flat_hbm, in_buf, out_buf, tail_v, si0, si1, so0, so1):
    wid = lax.axis_index("s") * NC + lax.axis_index("c")
    start = wid * COLS_LO + jnp.minimum(wid, COLS_EXTRA)
    ncols = COLS_LO + jnp.where(wid < COLS_EXTRA, 1, 0)
    last = start + ncols - 1
    lane = lax.iota(jnp.int32, 16)
    sin = (si0, si1)
    sout = (so0, so1)

    def col_of(i):
        return jnp.minimum(start + i, last)

    pltpu.async_copy(entT_hbm.at[:, pl.ds(col_of(0) * 128, 128)], in_buf.at[0], sin[0])

    def outer(h, _):
        for par in range(2):
            i = h * 2 + par
            pltpu.make_async_copy(entT_hbm.at[:, pl.ds(0, 128)], in_buf.at[par], sin[par]).wait()
            pltpu.async_copy(entT_hbm.at[:, pl.ds(col_of(i + 1) * 128, 128)],
                             in_buf.at[1 - par], sin[1 - par])
            @pl.when(i >= 2)
            def _():
                pltpu.make_async_copy(flat_hbm.at[pl.ds(0, 128 * DIM)],
                                      out_buf.at[par], sout[par]).wait()
            _transpose_block(in_buf, out_buf, par, lane, 128)
            pltpu.async_copy(out_buf.at[par],
                             flat_hbm.at[pl.ds(col_of(i) * (128 * DIM), 128 * DIM)],
                             sout[par])
        return 0

    lax.fori_loop(0, TRIP, outer, 0)

    pltpu.make_async_copy(flat_hbm.at[pl.ds(0, 128 * DIM)], out_buf.at[0], sout[0]).wait()
    pltpu.make_async_copy(flat_hbm.at[pl.ds(0, 128 * DIM)], out_buf.at[1], sout[1]).wait()
    pltpu.make_async_copy(entT_hbm.at[:, pl.ds(0, 128)], in_buf.at[0], sin[0]).wait()

    @pl.when(wid == NW - 1)
    def _():
        pltpu.sync_copy(tail_hbm, tail_v)
        pltpu.sync_copy(tail_v, flat_hbm.at[pl.ds(NCOL * 128 * DIM, TAIL * DIM)])


def _issue_group(g, bi, idx_s, idx_o, idx_r, flat_hbm, reltab_hbm,
                 sub_b, obj_b, rel_b, sems):
    vs = idx_s[pl.ds(g * GRP, GRP)]
    vo = idx_o[pl.ds(g * GRP, GRP)]
    vr = idx_r[pl.ds(g * GRP, GRP)]
    for l in range(GRP):
        row = pl.ds(l * DIM, DIM)
        pltpu.async_copy(flat_hbm.at[pl.ds(vs[l] * DIM, DIM)], sub_b.at[bi, row], sems[bi][0])
        pltpu.async_copy(flat_hbm.at[pl.ds(vo[l] * DIM, DIM)], obj_b.at[bi, row], sems[bi][1])
        pltpu.async_copy(reltab_hbm.at[pl.ds(vr[l], 1)], rel_b.at[bi, pl.ds(l, 1)], sems[bi][2])


def _gather_body(subj_hbm, obj_hbm, rel_hbm, flat_hbm, reltab_hbm, out_hbm,
                 idx_s, idx_o, idx_r, sub_b, obj_b, rel_b, score_v,
                 s0, s1, s2, s3, s4, s5, s6, s7, s8, s9, s10, s11):
    wid = lax.axis_index("s") * NC + lax.axis_index("c")
    sems = [(s0, s1, s2), (s3, s4, s5), (s6, s7, s8), (s9, s10, s11)]

    pltpu.sync_copy(subj_hbm.at[wid], idx_s)
    pltpu.sync_copy(obj_hbm.at[wid], idx_o)
    pltpu.sync_copy(rel_hbm.at[wid], idx_r)

    lane = lax.iota(jnp.int32, GRP)

    for g in range(NBUF - 1):
        _issue_group(g, g, idx_s, idx_o, idx_r, flat_hbm, reltab_hbm,
                     sub_b, obj_b, rel_b, sems)

    def outer(h, _):
        for p in range(NBUF):
            g = h * NBUF + p

            pltpu.make_async_copy(flat_hbm.at[pl.ds(0, GRP * DIM)],
                                  sub_b.at[p], sems[p][0]).wait()
            pltpu.make_async_copy(flat_hbm.at[pl.ds(0, GRP * DIM)],
                                  obj_b.at[p], sems[p][1]).wait()
            pltpu.make_async_copy(reltab_hbm.at[pl.ds(0, GRP)],
                                  rel_b.at[p], sems[p][2]).wait()

            nb = (p + NBUF - 1) % NBUF

            @pl.when(g + NBUF - 1 < NGRP)
            def _():
                _issue_group(g + NBUF - 1, nb, idx_s, idx_o, idx_r,
                             flat_hbm, reltab_hbm, sub_b, obj_b, rel_b, sems)

            out = jnp.zeros((GRP,), jnp.float32)
            for l in range(GRP):
                acc = jnp.zeros((16,), jnp.float32)
                for c in range(DIM // 16):
                    col = pl.ds(l * DIM + c * 16, 16)
                    d = sub_b[p, col] + rel_b[p, l, pl.ds(c * 16, 16)] - obj_b[p, col]
                    acc = acc + d * d
                s = lax.reduce_sum(acc, axes=(0,))
                out = jnp.where(lane == l, s, out)
            score_v[pl.ds(g * GRP, GRP)] = out
        return 0

    lax.fori_loop(0, NGRP // NBUF, outer, 0)

    pltpu.sync_copy(score_v, out_hbm.at[wid])


@jax.jit
def _transe(subjects, objects, relations, ent_embedding, rel_embedding):
    mesh = plsc.VectorSubcoreMesh(core_axis_name="c", subcore_axis_name="s")
    kern_t = pl.kernel(
        _tr_body,
        out_type=jax.ShapeDtypeStruct((FLAT,), jnp.float32),
        mesh=mesh,
        scratch_types=[
            pltpu.VMEM((2, DIM, 128), jnp.float32),
            pltpu.VMEM((2, 128 * DIM), jnp.float32),
            pltpu.VMEM((TAIL * DIM,), jnp.float32),
        ] + [pltpu.SemaphoreType.DMA] * 4,
        compiler_params=pltpu.CompilerParams(
            needs_layout_passes=False, use_tc_tiling_on_sc=True),
    )
    kern_g = pl.kernel(
        _gather_body,
        out_type=jax.ShapeDtypeStruct((NW, B_PER_W), jnp.float32),
        mesh=mesh,
        scratch_types=[
            pltpu.VMEM((B_PER_W,), jnp.int32),
            pltpu.VMEM((B_PER_W,), jnp.int32),
            pltpu.VMEM((B_PER_W,), jnp.int32),
            pltpu.VMEM((NBUF, GRP * DIM), jnp.float32),
            pltpu.VMEM((NBUF, GRP * DIM), jnp.float32),
            pltpu.VMEM((NBUF, GRP, DIM), jnp.float32),
            pltpu.VMEM((B_PER_W,), jnp.float32),
        ] + [pltpu.SemaphoreType.DMA] * 12,
        compiler_params=pltpu.CompilerParams(
            needs_layout_passes=False, use_tc_tiling_on_sc=True),
    )
    tail = ent_embedding[NCOL * 128:].reshape(TAIL * DIM)
    flat = kern_t(ent_embedding.T, tail)
    subj = subjects.astype(jnp.int32).reshape(NW, B_PER_W)
    obj = objects.astype(jnp.int32).reshape(NW, B_PER_W)
    rel = relations.astype(jnp.int32).reshape(NW, B_PER_W)
    out = kern_g(subj, obj, rel, flat, rel_embedding)
    return out.reshape(BATCH, 1)


def kernel(subjects, objects, relations, ent_embedding, rel_embedding):
    return _transe(subjects, objects, relations, ent_embedding, rel_embedding)


# speedup vs baseline: 1.3540x; 1.0519x over previous
import functools

import jax
import jax.numpy as jnp
from jax import lax
from jax.experimental import pallas as pl
from jax.experimental.pallas import tpu as pltpu
from jax.experimental.pallas import tpu_sc as plsc

NUM_ENT = 1000000
NUM_REL = 1000
DIM = 64
BATCH = 16384

NC = 2
NS = 16
NW = NC * NS
B_PER_W = BATCH // NW
GRP = 16
NGRP = B_PER_W // GRP
NBUF = 4

NCOL = NUM_ENT // 128
TAIL = NUM_ENT - NCOL * 128
FLAT = NUM_ENT * DIM
COLS_LO = NCOL // NW
COLS_EXTRA = NCOL - COLS_LO * NW
TRIP = 41


def _transpose_block(in_buf, out_buf, b, lane, n_ent):
    bfull = jnp.full((16,), b, jnp.int32)
    for half in range(n_ent // 128):
        hfull = jnp.full((16,), half, jnp.int32)
        for e0 in range(0, 128, 2):
            vals = []
            for e in (e0, e0 + 1):
                efull = jnp.full((16,), e, jnp.int32)
                for c in range(DIM // 16):
                    vals.append(plsc.load_gather(in_buf, [bfull, hfull, c * 16 + lane, efull]))
            for k, v in enumerate(vals):
                e = half * 128 + e0 + k // 4
                c = k % 4
                out_buf[pl.ds(b * (256 * DIM) + e * DIM + c * 16, 16)] = v


def _tr_body(entT_hbm, tail_hbm, flat_hbm, in_buf, out_buf, tail_v,
             si0, si1, si2, so0, so1, so2):
    wid = lax.axis_index("s") * NC + jnp.int32(lax.axis_index("c"))
    start = wid * COLS_LO + jnp.minimum(wid, COLS_EXTRA)
    ncols = COLS_LO + jnp.where(wid < COLS_EXTRA, 1, 0)
    lastb = start + ncols - 2
    lane = lax.iota(jnp.int32, 16)
    sin = (si0, si1, si2)
    sout = (so0, so1, so2)
    BW = 2 * 128
    BF = BW * DIM

    def col_of(b):
        return jnp.minimum(start + 2 * b, lastb)

    def issue_in(b, buf):
        c0 = col_of(b)
        pltpu.async_copy(entT_hbm.at[:, pl.ds(c0 * 128, 128)], in_buf.at[buf, 0], sin[buf])
        pltpu.async_copy(entT_hbm.at[:, pl.ds((c0 + 1) * 128, 128)], in_buf.at[buf, 1], sin[buf])

    issue_in(0, 0)
    issue_in(1, 1)

    def outer(h, _):
        for par in range(3):
            b = h * 3 + par
            pltpu.make_async_copy(entT_hbm.at[:, pl.ds(0, 128)], in_buf.at[par, 0], sin[par]).wait()
            pltpu.make_async_copy(entT_hbm.at[:, pl.ds(0, 128)], in_buf.at[par, 1], sin[par]).wait()
            issue_in(b + 2, (par + 2) % 3)

            @pl.when(b >= 3)
            def _():
                pltpu.make_async_copy(flat_hbm.at[pl.ds(0, BF)],
                                      out_buf.at[pl.ds(par * BF, BF)], sout[par]).wait()
            _transpose_block(in_buf, out_buf, par, lane, BW)
            pltpu.async_copy(out_buf.at[pl.ds(par * BF, BF)],
                             flat_hbm.at[pl.ds(col_of(b) * (128 * DIM), BF)],
                             sout[par])
        return 0

    lax.fori_loop(0, TRIP, outer, 0)

    for d in range(3):
        pltpu.make_async_copy(flat_hbm.at[pl.ds(0, BF)],
                              out_buf.at[pl.ds(d * BF, BF)], sout[d]).wait()
    for d in (3 * TRIP % 3, (3 * TRIP + 1) % 3):
        pltpu.make_async_copy(entT_hbm.at[:, pl.ds(0, 128)], in_buf.at[d, 0], sin[d]).wait()
        pltpu.make_async_copy(entT_hbm.at[:, pl.ds(0, 128)], in_buf.at[d, 1], sin[d]).wait()

    @pl.when(wid == NW - 1)
    def _():
        pltpu.sync_copy(tail_hbm, tail_v)
        pltpu.sync_copy(tail_v, flat_hbm.at[pl.ds(NCOL * 128 * DIM, TAIL * DIM)])


def _issue_group(g, bi, idx_s, idx_o, idx_r, flat_hbm, reltab_hbm,
                 sub_b, obj_b, rel_b, sems):
    vs = idx_s[pl.ds(g * GRP, GRP)]
    vo = idx_o[pl.ds(g * GRP, GRP)]
    vr = idx_r[pl.ds(g * GRP, GRP)]
    for l in range(GRP):
        row = pl.ds(l * DIM, DIM)
        pltpu.async_copy(flat_hbm.at[pl.ds(vs[l] * DIM, DIM)], sub_b.at[bi, row], sems[bi][0])
        pltpu.async_copy(flat_hbm.at[pl.ds(vo[l] * DIM, DIM)], obj_b.at[bi, row], sems[bi][1])
        pltpu.async_copy(reltab_hbm.at[pl.ds(vr[l], 1)], rel_b.at[bi, pl.ds(l, 1)], sems[bi][2])


def _gather_body(subj_hbm, obj_hbm, rel_hbm, flat_hbm, reltab_hbm, out_hbm,
                 idx_s, idx_o, idx_r, sub_b, obj_b, rel_b, score_v,
                 s0, s1, s2, s3, s4, s5, s6, s7, s8, s9, s10, s11):
    wid = lax.axis_index("s") * NC + lax.axis_index("c")
    sems = [(s0, s1, s2), (s3, s4, s5), (s6, s7, s8), (s9, s10, s11)]

    pltpu.sync_copy(subj_hbm.at[wid], idx_s)
    pltpu.sync_copy(obj_hbm.at[wid], idx_o)
    pltpu.sync_copy(rel_hbm.at[wid], idx_r)

    lane = lax.iota(jnp.int32, GRP)

    for g in range(NBUF - 1):
        _issue_group(g, g, idx_s, idx_o, idx_r, flat_hbm, reltab_hbm,
                     sub_b, obj_b, rel_b, sems)

    def outer(h, _):
        for p in range(NBUF):
            g = h * NBUF + p

            pltpu.make_async_copy(flat_hbm.at[pl.ds(0, GRP * DIM)],
                                  sub_b.at[p], sems[p][0]).wait()
            pltpu.make_async_copy(flat_hbm.at[pl.ds(0, GRP * DIM)],
                                  obj_b.at[p], sems[p][1]).wait()
            pltpu.make_async_copy(reltab_hbm.at[pl.ds(0, GRP)],
                                  rel_b.at[p], sems[p][2]).wait()

            nb = (p + NBUF - 1) % NBUF

            @pl.when(g + NBUF - 1 < NGRP)
            def _():
                _issue_group(g + NBUF - 1, nb, idx_s, idx_o, idx_r,
                             flat_hbm, reltab_hbm, sub_b, obj_b, rel_b, sems)

            out = jnp.zeros((GRP,), jnp.float32)
            for l in range(GRP):
                acc = jnp.zeros((16,), jnp.float32)
                for c in range(DIM // 16):
                    col = pl.ds(l * DIM + c * 16, 16)
                    d = sub_b[p, col] + rel_b[p, l, pl.ds(c * 16, 16)] - obj_b[p, col]
                    acc = acc + d * d
                s = lax.reduce_sum(acc, axes=(0,))
                out = jnp.where(lane == l, s, out)
            score_v[pl.ds(g * GRP, GRP)] = out
        return 0

    lax.fori_loop(0, NGRP // NBUF, outer, 0)

    pltpu.sync_copy(score_v, out_hbm.at[wid])


@jax.jit
def _transe(subjects, objects, relations, ent_embedding, rel_embedding):
    mesh = plsc.VectorSubcoreMesh(core_axis_name="c", subcore_axis_name="s")
    kern_t = pl.kernel(
        _tr_body,
        out_type=jax.ShapeDtypeStruct((FLAT,), jnp.float32),
        mesh=mesh,
        scratch_types=[
            pltpu.VMEM((3, 2, DIM, 128), jnp.float32),
            pltpu.VMEM((3 * 256 * DIM,), jnp.float32),
            pltpu.VMEM((TAIL * DIM,), jnp.float32),
        ] + [pltpu.SemaphoreType.DMA] * 6,
        compiler_params=pltpu.CompilerParams(
            needs_layout_passes=False, use_tc_tiling_on_sc=True),
    )
    kern_g = pl.kernel(
        _gather_body,
        out_type=jax.ShapeDtypeStruct((NW, B_PER_W), jnp.float32),
        mesh=mesh,
        scratch_types=[
            pltpu.VMEM((B_PER_W,), jnp.int32),
            pltpu.VMEM((B_PER_W,), jnp.int32),
            pltpu.VMEM((B_PER_W,), jnp.int32),
            pltpu.VMEM((NBUF, GRP * DIM), jnp.float32),
            pltpu.VMEM((NBUF, GRP * DIM), jnp.float32),
            pltpu.VMEM((NBUF, GRP, DIM), jnp.float32),
            pltpu.VMEM((B_PER_W,), jnp.float32),
        ] + [pltpu.SemaphoreType.DMA] * 12,
        compiler_params=pltpu.CompilerParams(
            needs_layout_passes=False, use_tc_tiling_on_sc=True),
    )
    tail = ent_embedding[NCOL * 128:].reshape(TAIL * DIM)
    flat = kern_t(ent_embedding.T, tail)
    subj = subjects.astype(jnp.int32).reshape(NW, B_PER_W)
    obj = objects.astype(jnp.int32).reshape(NW, B_PER_W)
    rel = relations.astype(jnp.int32).reshape(NW, B_PER_W)
    out = kern_g(subj, obj, rel, flat, rel_embedding)
    return out.reshape(BATCH, 1)


def kernel(subjects, objects, relations, ent_embedding, rel_embedding):
    return _transe(subjects, objects, relations, ent_embedding, rel_embedding)


# speedup vs baseline: 4.2486x; 3.1379x over previous
import functools

import jax
import jax.numpy as jnp
from jax import lax
from jax.experimental import pallas as pl
from jax.experimental.pallas import tpu as pltpu
from jax.experimental.pallas import tpu_sc as plsc

NUM_ENT = 1000000
NUM_REL = 1000
DIM = 64
BATCH = 16384

NC = 2
NS = 16
NW = NC * NS
B_PER_W = BATCH // NW
GRP = 16
NGRP = B_PER_W // GRP
NBUF = 4


def _issue_group(g, bi, idx_s, idx_o, idx_r, ent_hbm, reltab_hbm,
                 sub_b, obj_b, rel_b, sems):
    vs = idx_s[pl.ds(g * GRP, GRP)]
    vo = idx_o[pl.ds(g * GRP, GRP)]
    vr = idx_r[pl.ds(g * GRP, GRP)]
    for l in range(GRP):
        row = pl.ds(l, 1)
        pltpu.async_copy(ent_hbm.at[pl.ds(vs[l], 1)], sub_b.at[bi, row], sems[bi][0])
        pltpu.async_copy(ent_hbm.at[pl.ds(vo[l], 1)], obj_b.at[bi, row], sems[bi][1])
        pltpu.async_copy(reltab_hbm.at[pl.ds(vr[l], 1)], rel_b.at[bi, row], sems[bi][2])


def _transe_body(subj_hbm, obj_hbm, rel_hbm, ent_hbm, reltab_hbm, out_hbm,
                 idx_s, idx_o, idx_r, sub_b, obj_b, rel_b, score_v,
                 s0, s1, s2, s3, s4, s5, s6, s7, s8, s9, s10, s11):
    wid = lax.axis_index("s") * NC + lax.axis_index("c")
    sems = [(s0, s1, s2), (s3, s4, s5), (s6, s7, s8), (s9, s10, s11)]

    pltpu.sync_copy(subj_hbm.at[wid], idx_s)
    pltpu.sync_copy(obj_hbm.at[wid], idx_o)
    pltpu.sync_copy(rel_hbm.at[wid], idx_r)

    lane = lax.iota(jnp.int32, GRP)

    for g in range(NBUF - 1):
        _issue_group(g, g, idx_s, idx_o, idx_r, ent_hbm, reltab_hbm,
                     sub_b, obj_b, rel_b, sems)

    def outer(h, _):
        for p in range(NBUF):
            g = h * NBUF + p

            pltpu.make_async_copy(ent_hbm.at[pl.ds(0, GRP)], sub_b.at[p], sems[p][0]).wait()
            pltpu.make_async_copy(ent_hbm.at[pl.ds(0, GRP)], obj_b.at[p], sems[p][1]).wait()
            pltpu.make_async_copy(reltab_hbm.at[pl.ds(0, GRP)], rel_b.at[p], sems[p][2]).wait()

            nb = (p + NBUF - 1) % NBUF

            @pl.when(g + NBUF - 1 < NGRP)
            def _():
                _issue_group(g + NBUF - 1, nb, idx_s, idx_o, idx_r,
                             ent_hbm, reltab_hbm, sub_b, obj_b, rel_b, sems)

            out = jnp.zeros((GRP,), jnp.float32)
            for l in range(GRP):
                acc = jnp.zeros((16,), jnp.float32)
                for c in range(DIM // 16):
                    col = pl.ds(c * 16, 16)
                    d = sub_b[p, l, col] + rel_b[p, l, col] - obj_b[p, l, col]
                    acc = acc + d * d
                s = lax.reduce_sum(acc, axes=(0,))
                out = jnp.where(lane == l, s, out)
            score_v[pl.ds(g * GRP, GRP)] = out
        return 0

    lax.fori_loop(0, NGRP // NBUF, outer, 0)

    pltpu.sync_copy(score_v, out_hbm.at[wid])


@jax.jit
def _transe(subjects, objects, relations, ent_embedding, rel_embedding):
    mesh = plsc.VectorSubcoreMesh(core_axis_name="c", subcore_axis_name="s")
    kern = pl.kernel(
        _transe_body,
        out_type=jax.ShapeDtypeStruct((NW, B_PER_W), jnp.float32),
        mesh=mesh,
        scratch_types=[
            pltpu.VMEM((B_PER_W,), jnp.int32),
            pltpu.VMEM((B_PER_W,), jnp.int32),
            pltpu.VMEM((B_PER_W,), jnp.int32),
            pltpu.VMEM((NBUF, GRP, DIM), jnp.float32),
            pltpu.VMEM((NBUF, GRP, DIM), jnp.float32),
            pltpu.VMEM((NBUF, GRP, DIM), jnp.float32),
            pltpu.VMEM((B_PER_W,), jnp.float32),
        ] + [pltpu.SemaphoreType.DMA] * 12,
        compiler_params=pltpu.CompilerParams(
            needs_layout_passes=False, use_tc_tiling_on_sc=True),
    )
    subj = subjects.astype(jnp.int32).reshape(NW, B_PER_W)
    obj = objects.astype(jnp.int32).reshape(NW, B_PER_W)
    rel = relations.astype(jnp.int32).reshape(NW, B_PER_W)
    out = kern(subj, obj, rel, ent_embedding, rel_embedding)
    return out.reshape(BATCH, 1)


def kernel(subjects, objects, relations, ent_embedding, rel_embedding):
    return _transe(subjects, objects, relations, ent_embedding, rel_embedding)
